# baseline (device time: 767057 ns/iter reference)
import jax
import jax.numpy as jnp
from jax import lax
from jax.experimental import pallas as pl
from jax.experimental.pallas import tpu as pltpu

N_DEV = 16
GELU_C = 0.7978845608028654


def kernel(x, w_mat):
    m_global, k_per = x.shape
    _, n = w_mat.shape
    m_per = m_global // N_DEV

    def body(x_ref, w_ref, out_ref, comm_ref, send_sems, recv_sems):
        my = lax.axis_index("i")
        left = lax.rem(my - 1 + N_DEV, N_DEV)
        right = lax.rem(my + 1, N_DEV)

        barrier_sem = pltpu.get_barrier_semaphore()
        for nbr in (left, right):
            pl.semaphore_signal(
                barrier_sem, inc=1,
                device_id=(nbr,), device_id_type=pl.DeviceIdType.MESH,
            )
        pl.semaphore_wait(barrier_sem, 2)

        def partial_chunk(c):
            xc = x_ref[pl.ds(c * m_per, m_per), :]
            return jnp.dot(xc, w_ref[:, :], preferred_element_type=jnp.float32)

        c0 = lax.rem(my - 1 + N_DEV, N_DEV)
        comm_ref[0] = partial_chunk(c0).astype(jnp.bfloat16)

        for s in range(N_DEV - 1):
            send_slot = s % 2
            recv_slot = (s + 1) % 2
            rdma = pltpu.make_async_remote_copy(
                src_ref=comm_ref.at[send_slot],
                dst_ref=comm_ref.at[recv_slot],
                send_sem=send_sems.at[send_slot],
                recv_sem=recv_sems.at[recv_slot],
                device_id=(right,),
                device_id_type=pl.DeviceIdType.MESH,
            )
            rdma.start()
            rdma.wait()

            r = lax.rem(my - 2 - s + 2 * N_DEV, N_DEV)
            acc = comm_ref[recv_slot].astype(jnp.float32) + partial_chunk(r)
            if s < N_DEV - 2:
                comm_ref[recv_slot] = acc.astype(jnp.bfloat16)
            else:
                y = acc
                out_ref[:, :] = 0.5 * y * (
                    1.0 + jnp.tanh(GELU_C * (y + 0.044715 * y * y * y))
                )

    return pl.pallas_call(
        body,
        out_shape=jax.ShapeDtypeStruct((m_per, n), jnp.float32),
        in_specs=[
            pl.BlockSpec(memory_space=pltpu.VMEM),
            pl.BlockSpec(memory_space=pltpu.VMEM),
        ],
        out_specs=pl.BlockSpec(memory_space=pltpu.VMEM),
        scratch_shapes=[
            pltpu.VMEM((2, m_per, n), jnp.bfloat16),
            pltpu.SemaphoreType.DMA((2,)),
            pltpu.SemaphoreType.DMA((2,)),
        ],
        compiler_params=pltpu.CompilerParams(collective_id=0),
    )(x, w_mat)


# device time: 434196 ns/iter; 1.7666x vs baseline; 1.7666x over previous
import jax
import jax.numpy as jnp
from jax import lax
from jax.experimental import pallas as pl
from jax.experimental.pallas import tpu as pltpu

N_DEV = 16
GELU_C = 0.7978845608028654


def _gelu(y):
    return 0.5 * y * (1.0 + jnp.tanh(GELU_C * (y + 0.044715 * y * y * y)))


def kernel(x, w_mat):
    m_global, k_per = x.shape
    _, n = w_mat.shape
    m_per = m_global // N_DEV
    nh = n // 2

    def body(x_ref, w_ref, out_ref, cw_ref, ccw_ref,
             cw_ssem, cw_rsem, ccw_ssem, ccw_rsem):
        my = lax.axis_index("i")
        left = lax.rem(my - 1 + N_DEV, N_DEV)
        right = lax.rem(my + 1, N_DEV)

        barrier_sem = pltpu.get_barrier_semaphore()
        for nbr in (left, right):
            pl.semaphore_signal(
                barrier_sem, inc=1,
                device_id=(nbr,), device_id_type=pl.DeviceIdType.MESH,
            )
        pl.semaphore_wait(barrier_sem, 2)

        def pc_cw(c):
            xc = x_ref[pl.ds(c * m_per, m_per), :]
            return jnp.dot(xc, w_ref[:, :nh], preferred_element_type=jnp.float32)

        def pc_ccw(c):
            xc = x_ref[pl.ds(c * m_per, m_per), :]
            return jnp.dot(xc, w_ref[:, nh:], preferred_element_type=jnp.float32)

        cw_ref[0] = pc_cw(lax.rem(my - 1 + N_DEV, N_DEV)).astype(jnp.bfloat16)
        ccw_ref[0] = pc_ccw(lax.rem(my + 1, N_DEV)).astype(jnp.bfloat16)

        for s in range(N_DEV - 1):
            ss = s % 2
            rs = (s + 1) % 2
            cw = pltpu.make_async_remote_copy(
                src_ref=cw_ref.at[ss], dst_ref=cw_ref.at[rs],
                send_sem=cw_ssem.at[ss], recv_sem=cw_rsem.at[rs],
                device_id=(right,), device_id_type=pl.DeviceIdType.MESH,
            )
            ccw = pltpu.make_async_remote_copy(
                src_ref=ccw_ref.at[ss], dst_ref=ccw_ref.at[rs],
                send_sem=ccw_ssem.at[ss], recv_sem=ccw_rsem.at[rs],
                device_id=(left,), device_id_type=pl.DeviceIdType.MESH,
            )
            cw.start()
            ccw.start()

            r_cw = lax.rem(my - 2 - s + 2 * N_DEV, N_DEV)
            r_ccw = lax.rem(my + 2 + s, N_DEV)
            p_cw = pc_cw(r_cw)
            p_ccw = pc_ccw(r_ccw)

            cw.wait_recv()
            ccw.wait_recv()
            acc_cw = cw_ref[rs].astype(jnp.float32) + p_cw
            acc_ccw = ccw_ref[rs].astype(jnp.float32) + p_ccw
            cw.wait_send()
            ccw.wait_send()
            if s < N_DEV - 2:
                cw_ref[rs] = acc_cw.astype(jnp.bfloat16)
                ccw_ref[rs] = acc_ccw.astype(jnp.bfloat16)
            else:
                out_ref[:, :nh] = _gelu(acc_cw)
                out_ref[:, nh:] = _gelu(acc_ccw)

    return pl.pallas_call(
        body,
        out_shape=jax.ShapeDtypeStruct((m_per, n), jnp.float32),
        in_specs=[
            pl.BlockSpec(memory_space=pltpu.VMEM),
            pl.BlockSpec(memory_space=pltpu.VMEM),
        ],
        out_specs=pl.BlockSpec(memory_space=pltpu.VMEM),
        scratch_shapes=[
            pltpu.VMEM((2, m_per, nh), jnp.bfloat16),
            pltpu.VMEM((2, m_per, nh), jnp.bfloat16),
            pltpu.SemaphoreType.DMA((2,)),
            pltpu.SemaphoreType.DMA((2,)),
            pltpu.SemaphoreType.DMA((2,)),
            pltpu.SemaphoreType.DMA((2,)),
        ],
        compiler_params=pltpu.CompilerParams(
            collective_id=0,
            vmem_limit_bytes=100 * 1024 * 1024,
        ),
    )(x, w_mat)


# device time: 412881 ns/iter; 1.8578x vs baseline; 1.0516x over previous
import jax
import jax.numpy as jnp
from jax import lax
from jax.experimental import pallas as pl
from jax.experimental.pallas import tpu as pltpu

N_DEV = 16
GELU_C = 0.7978845608028654

RING = [0, 3, 7, 4, 8, 11, 15, 12, 13, 14, 10, 9, 5, 6, 2, 1]
POS = [RING.index(i) for i in range(N_DEV)]


def _lut(idx, table):
    v = jnp.int32(table[0])
    for i in range(1, N_DEV):
        v = jnp.where(idx == i, jnp.int32(table[i]), v)
    return v


def _gelu(y):
    return 0.5 * y * (1.0 + jnp.tanh(GELU_C * (y + 0.044715 * y * y * y)))


def kernel(x, w_mat):
    m_global, k_per = x.shape
    _, n = w_mat.shape
    m_per = m_global // N_DEV
    nh = n // 2

    def body(x_ref, w_ref, out_ref, cw_ref, ccw_ref,
             cw_ssem, cw_rsem, ccw_ssem, ccw_rsem):
        my = lax.axis_index("i")
        k = _lut(my, POS)
        right = _lut(lax.rem(k + 1, N_DEV), RING)
        left = _lut(lax.rem(k - 1 + N_DEV, N_DEV), RING)

        barrier_sem = pltpu.get_barrier_semaphore()
        for nbr in (left, right):
            pl.semaphore_signal(
                barrier_sem, inc=1,
                device_id=(nbr,), device_id_type=pl.DeviceIdType.MESH,
            )
        pl.semaphore_wait(barrier_sem, 2)

        def pc_cw(c):
            xc = x_ref[pl.ds(c * m_per, m_per), :]
            return jnp.dot(xc, w_ref[:, :nh], preferred_element_type=jnp.float32)

        def pc_ccw(c):
            xc = x_ref[pl.ds(c * m_per, m_per), :]
            return jnp.dot(xc, w_ref[:, nh:], preferred_element_type=jnp.float32)

        cw_ref[0] = pc_cw(left).astype(jnp.bfloat16)
        ccw_ref[0] = pc_ccw(right).astype(jnp.bfloat16)

        for s in range(N_DEV - 1):
            ss = s % 2
            rs = (s + 1) % 2
            cw = pltpu.make_async_remote_copy(
                src_ref=cw_ref.at[ss], dst_ref=cw_ref.at[rs],
                send_sem=cw_ssem.at[ss], recv_sem=cw_rsem.at[rs],
                device_id=(right,), device_id_type=pl.DeviceIdType.MESH,
            )
            ccw = pltpu.make_async_remote_copy(
                src_ref=ccw_ref.at[ss], dst_ref=ccw_ref.at[rs],
                send_sem=ccw_ssem.at[ss], recv_sem=ccw_rsem.at[rs],
                device_id=(left,), device_id_type=pl.DeviceIdType.MESH,
            )
            cw.start()
            ccw.start()

            r_cw = _lut(lax.rem(k - 2 - s + 2 * N_DEV, N_DEV), RING)
            r_ccw = _lut(lax.rem(k + 2 + s, N_DEV), RING)
            p_cw = pc_cw(r_cw)
            p_ccw = pc_ccw(r_ccw)

            cw.wait_recv()
            ccw.wait_recv()
            acc_cw = cw_ref[rs].astype(jnp.float32) + p_cw
            acc_ccw = ccw_ref[rs].astype(jnp.float32) + p_ccw
            cw.wait_send()
            ccw.wait_send()
            if s < N_DEV - 2:
                cw_ref[rs] = acc_cw.astype(jnp.bfloat16)
                ccw_ref[rs] = acc_ccw.astype(jnp.bfloat16)
            else:
                out_ref[:, :nh] = _gelu(acc_cw)
                out_ref[:, nh:] = _gelu(acc_ccw)

    return pl.pallas_call(
        body,
        out_shape=jax.ShapeDtypeStruct((m_per, n), jnp.float32),
        in_specs=[
            pl.BlockSpec(memory_space=pltpu.VMEM),
            pl.BlockSpec(memory_space=pltpu.VMEM),
        ],
        out_specs=pl.BlockSpec(memory_space=pltpu.VMEM),
        scratch_shapes=[
            pltpu.VMEM((2, m_per, nh), jnp.bfloat16),
            pltpu.VMEM((2, m_per, nh), jnp.bfloat16),
            pltpu.SemaphoreType.DMA((2,)),
            pltpu.SemaphoreType.DMA((2,)),
            pltpu.SemaphoreType.DMA((2,)),
            pltpu.SemaphoreType.DMA((2,)),
        ],
        compiler_params=pltpu.CompilerParams(
            collective_id=0,
            vmem_limit_bytes=100 * 1024 * 1024,
        ),
    )(x, w_mat)


# device time: 363581 ns/iter; 2.1097x vs baseline; 1.1356x over previous
import jax
import jax.numpy as jnp
from jax import lax
from jax.experimental import pallas as pl
from jax.experimental.pallas import tpu as pltpu

N_DEV = 16
GELU_C = 0.7978845608028654

RING = [0, 3, 7, 4, 8, 11, 15, 12, 13, 14, 10, 9, 5, 6, 2, 1]
POS = [RING.index(i) for i in range(N_DEV)]


def _lut(idx, table):
    v = jnp.int32(table[0])
    for i in range(1, N_DEV):
        v = jnp.where(idx == i, jnp.int32(table[i]), v)
    return v


def _gelu(y):
    return 0.5 * y * (1.0 + jnp.tanh(GELU_C * (y + 0.044715 * y * y * y)))


def kernel(x, w_mat):
    m_global, k_per = x.shape
    _, n = w_mat.shape
    m_per = m_global // N_DEV
    nq = n // 4

    def body(x_ref, w_ref, out_ref,
             cw0_ref, cw1_ref, ccw0_ref, ccw1_ref,
             cw0_ssem, cw0_rsem, cw1_ssem, cw1_rsem,
             ccw0_ssem, ccw0_rsem, ccw1_ssem, ccw1_rsem):
        my = lax.axis_index("i")
        k = _lut(my, POS)
        right = _lut(lax.rem(k + 1, N_DEV), RING)
        left = _lut(lax.rem(k - 1 + N_DEV, N_DEV), RING)

        barrier_sem = pltpu.get_barrier_semaphore()
        for nbr in (left, right):
            pl.semaphore_signal(
                barrier_sem, inc=1,
                device_id=(nbr,), device_id_type=pl.DeviceIdType.MESH,
            )
        pl.semaphore_wait(barrier_sem, 2)

        def pc(c, lo):
            xc = x_ref[pl.ds(c * m_per, m_per), :]
            return jnp.dot(
                xc, w_ref[:, lo:lo + nq], preferred_element_type=jnp.float32
            )

        chains = [
            (cw0_ref, cw0_ssem, cw0_rsem, right, 0 * nq, True),
            (ccw0_ref, ccw0_ssem, ccw0_rsem, left, 2 * nq, False),
            (cw1_ref, cw1_ssem, cw1_rsem, right, 1 * nq, True),
            (ccw1_ref, ccw1_ssem, ccw1_rsem, left, 3 * nq, False),
        ]

        def arriving(s, is_cw):
            if is_cw:
                return _lut(lax.rem(k - 2 - s + 2 * N_DEV, N_DEV), RING)
            return _lut(lax.rem(k + 2 + s, N_DEV), RING)

        inflight = {}
        prevprev = {}
        for buf, ssem, rsem, tgt, lo, is_cw in chains:
            first = left if is_cw else right
            buf[0] = pc(first, lo).astype(jnp.bfloat16)
        for ci, (buf, ssem, rsem, tgt, lo, is_cw) in enumerate(chains):
            rdma = pltpu.make_async_remote_copy(
                src_ref=buf.at[0], dst_ref=buf.at[1],
                send_sem=ssem.at[0], recv_sem=rsem.at[1],
                device_id=(tgt,), device_id_type=pl.DeviceIdType.MESH,
            )
            rdma.start()
            inflight[ci] = rdma
            prevprev[ci] = None

        for s in range(N_DEV - 1):
            ss = s % 2
            rs = (s + 1) % 2
            p = {}
            for ci, (buf, ssem, rsem, tgt, lo, is_cw) in enumerate(chains):
                p[ci] = pc(arriving(s, is_cw), lo)
            for ci, (buf, ssem, rsem, tgt, lo, is_cw) in enumerate(chains):
                cur = inflight[ci]
                cur.wait_recv()
                if prevprev[ci] is not None:
                    prevprev[ci].wait_send()
                acc = buf[rs].astype(jnp.float32) + p[ci]
                if s < N_DEV - 2:
                    buf[rs] = acc.astype(jnp.bfloat16)
                    nxt = pltpu.make_async_remote_copy(
                        src_ref=buf.at[rs], dst_ref=buf.at[ss],
                        send_sem=ssem.at[rs], recv_sem=rsem.at[ss],
                        device_id=(tgt,), device_id_type=pl.DeviceIdType.MESH,
                    )
                    nxt.start()
                    prevprev[ci] = cur
                    inflight[ci] = nxt
                else:
                    out_ref[:, lo:lo + nq] = _gelu(acc)

        for ci in range(len(chains)):
            inflight[ci].wait_send()

    return pl.pallas_call(
        body,
        out_shape=jax.ShapeDtypeStruct((m_per, n), jnp.float32),
        in_specs=[
            pl.BlockSpec(memory_space=pltpu.VMEM),
            pl.BlockSpec(memory_space=pltpu.VMEM),
        ],
        out_specs=pl.BlockSpec(memory_space=pltpu.VMEM),
        scratch_shapes=[
            pltpu.VMEM((2, m_per, nq), jnp.bfloat16),
            pltpu.VMEM((2, m_per, nq), jnp.bfloat16),
            pltpu.VMEM((2, m_per, nq), jnp.bfloat16),
            pltpu.VMEM((2, m_per, nq), jnp.bfloat16),
            pltpu.SemaphoreType.DMA((2,)),
            pltpu.SemaphoreType.DMA((2,)),
            pltpu.SemaphoreType.DMA((2,)),
            pltpu.SemaphoreType.DMA((2,)),
            pltpu.SemaphoreType.DMA((2,)),
            pltpu.SemaphoreType.DMA((2,)),
            pltpu.SemaphoreType.DMA((2,)),
            pltpu.SemaphoreType.DMA((2,)),
        ],
        compiler_params=pltpu.CompilerParams(
            collective_id=0,
            vmem_limit_bytes=100 * 1024 * 1024,
        ),
    )(x, w_mat)


# device time: 361996 ns/iter; 2.1190x vs baseline; 1.0044x over previous
import jax
import jax.numpy as jnp
from jax import lax
from jax.experimental import pallas as pl
from jax.experimental.pallas import tpu as pltpu

N_DEV = 16
GELU_C = 0.7978845608028654

RING = [0, 3, 7, 4, 8, 11, 15, 12, 13, 14, 10, 9, 5, 6, 2, 1]
POS = [RING.index(i) for i in range(N_DEV)]


def _lut(idx, table):
    v = jnp.int32(table[0])
    for i in range(1, N_DEV):
        v = jnp.where(idx == i, jnp.int32(table[i]), v)
    return v


def _gelu(y):
    return 0.5 * y * (1.0 + jnp.tanh(GELU_C * (y + 0.044715 * y * y * y)))


def kernel(x, w_mat):
    m_global, k_per = x.shape
    _, n = w_mat.shape
    m_per = m_global // N_DEV
    nq = n // 4

    def body(x_ref, w_ref, out_ref,
             cw0_ref, cw1_ref, ccw0_ref, ccw1_ref,
             cw0_ssem, cw0_rsem, cw1_ssem, cw1_rsem,
             ccw0_ssem, ccw0_rsem, ccw1_ssem, ccw1_rsem):
        my = lax.axis_index("i")
        k = _lut(my, POS)
        right = _lut(lax.rem(k + 1, N_DEV), RING)
        left = _lut(lax.rem(k - 1 + N_DEV, N_DEV), RING)

        barrier_sem = pltpu.get_barrier_semaphore()
        for nbr in (left, right):
            pl.semaphore_signal(
                barrier_sem, inc=1,
                device_id=(nbr,), device_id_type=pl.DeviceIdType.MESH,
            )
        pl.semaphore_wait(barrier_sem, 2)

        def pc(c, lo):
            xc = x_ref[pl.ds(c * m_per, m_per), :]
            return jnp.dot(
                xc, w_ref[:, lo:lo + nq], preferred_element_type=jnp.float32
            )

        chains = [
            (cw0_ref, cw0_ssem, cw0_rsem, right, 0 * nq, True),
            (ccw0_ref, ccw0_ssem, ccw0_rsem, left, 2 * nq, False),
            (cw1_ref, cw1_ssem, cw1_rsem, right, 1 * nq, True),
            (ccw1_ref, ccw1_ssem, ccw1_rsem, left, 3 * nq, False),
        ]

        def arriving(s, is_cw):
            if is_cw:
                return _lut(lax.rem(k - 2 - s + 2 * N_DEV, N_DEV), RING)
            return _lut(lax.rem(k + 2 + s, N_DEV), RING)

        inflight = {}
        prevprev = {}
        for ci, (buf, ssem, rsem, tgt, lo, is_cw) in enumerate(chains):
            first = left if is_cw else right
            buf[0] = pc(first, lo).astype(jnp.bfloat16)
            rdma = pltpu.make_async_remote_copy(
                src_ref=buf.at[0], dst_ref=buf.at[1],
                send_sem=ssem.at[0], recv_sem=rsem.at[1],
                device_id=(tgt,), device_id_type=pl.DeviceIdType.MESH,
            )
            rdma.start()
            inflight[ci] = rdma
            prevprev[ci] = None

        for s in range(N_DEV - 1):
            ss = s % 2
            rs = (s + 1) % 2
            p = {}
            for ci, (buf, ssem, rsem, tgt, lo, is_cw) in enumerate(chains):
                p[ci] = pc(arriving(s, is_cw), lo)
            for ci, (buf, ssem, rsem, tgt, lo, is_cw) in enumerate(chains):
                cur = inflight[ci]
                cur.wait_recv()
                if prevprev[ci] is not None:
                    prevprev[ci].wait_send()
                acc = buf[rs].astype(jnp.float32) + p[ci]
                if s < N_DEV - 2:
                    buf[rs] = acc.astype(jnp.bfloat16)
                    nxt = pltpu.make_async_remote_copy(
                        src_ref=buf.at[rs], dst_ref=buf.at[ss],
                        send_sem=ssem.at[rs], recv_sem=rsem.at[ss],
                        device_id=(tgt,), device_id_type=pl.DeviceIdType.MESH,
                    )
                    nxt.start()
                    prevprev[ci] = cur
                    inflight[ci] = nxt
                else:
                    out_ref[:, lo:lo + nq] = _gelu(acc)

        for ci in range(len(chains)):
            inflight[ci].wait_send()

    return pl.pallas_call(
        body,
        out_shape=jax.ShapeDtypeStruct((m_per, n), jnp.float32),
        in_specs=[
            pl.BlockSpec(memory_space=pltpu.VMEM),
            pl.BlockSpec(memory_space=pltpu.VMEM),
        ],
        out_specs=pl.BlockSpec(memory_space=pltpu.VMEM),
        scratch_shapes=[
            pltpu.VMEM((2, m_per, nq), jnp.bfloat16),
            pltpu.VMEM((2, m_per, nq), jnp.bfloat16),
            pltpu.VMEM((2, m_per, nq), jnp.bfloat16),
            pltpu.VMEM((2, m_per, nq), jnp.bfloat16),
            pltpu.SemaphoreType.DMA((2,)),
            pltpu.SemaphoreType.DMA((2,)),
            pltpu.SemaphoreType.DMA((2,)),
            pltpu.SemaphoreType.DMA((2,)),
            pltpu.SemaphoreType.DMA((2,)),
            pltpu.SemaphoreType.DMA((2,)),
            pltpu.SemaphoreType.DMA((2,)),
            pltpu.SemaphoreType.DMA((2,)),
        ],
        compiler_params=pltpu.CompilerParams(
            collective_id=0,
            vmem_limit_bytes=100 * 1024 * 1024,
        ),
    )(x, w_mat)


# device time: 361179 ns/iter; 2.1238x vs baseline; 1.0023x over previous
import jax
import jax.numpy as jnp
from jax import lax
from jax.experimental import pallas as pl
from jax.experimental.pallas import tpu as pltpu

N_DEV = 16
Q = 4
GELU_C = 0.7978845608028654

RING = [0, 3, 7, 4, 8, 11, 15, 12, 13, 14, 10, 9, 5, 6, 2, 1]
POS = [RING.index(i) for i in range(N_DEV)]


def _lut(idx, table):
    v = jnp.int32(table[0])
    for i in range(1, N_DEV):
        v = jnp.where(idx == i, jnp.int32(table[i]), v)
    return v


def _gelu(y):
    return 0.5 * y * (1.0 + jnp.tanh(GELU_C * (y + 0.044715 * y * y * y)))


def kernel(x, w_mat):
    m_global, k_per = x.shape
    _, n = w_mat.shape
    m_per = m_global // N_DEV
    ns = n // (2 * Q)

    def body(x_ref, w_ref, out_ref, *scratch):
        bufs = scratch[:2 * Q]
        sems = scratch[2 * Q:]
        my = lax.axis_index("i")
        k = _lut(my, POS)
        right = _lut(lax.rem(k + 1, N_DEV), RING)
        left = _lut(lax.rem(k - 1 + N_DEV, N_DEV), RING)

        barrier_sem = pltpu.get_barrier_semaphore()
        for nbr in (left, right):
            pl.semaphore_signal(
                barrier_sem, inc=1,
                device_id=(nbr,), device_id_type=pl.DeviceIdType.MESH,
            )
        pl.semaphore_wait(barrier_sem, 2)

        def pc(c, lo):
            xc = x_ref[pl.ds(c * m_per, m_per), :]
            return jnp.dot(
                xc, w_ref[:, lo:lo + ns], preferred_element_type=jnp.float32
            )

        chains = []
        for q in range(Q):
            chains.append(
                (bufs[2 * q], sems[4 * q], sems[4 * q + 1],
                 right, q * ns, True)
            )
            chains.append(
                (bufs[2 * q + 1], sems[4 * q + 2], sems[4 * q + 3],
                 left, (Q + q) * ns, False)
            )

        def arriving(s, is_cw):
            if is_cw:
                return _lut(lax.rem(k - 2 - s + 2 * N_DEV, N_DEV), RING)
            return _lut(lax.rem(k + 2 + s, N_DEV), RING)

        inflight = {}
        prevprev = {}
        for ci, (buf, ssem, rsem, tgt, lo, is_cw) in enumerate(chains):
            first = left if is_cw else right
            buf[0] = pc(first, lo).astype(jnp.bfloat16)
            rdma = pltpu.make_async_remote_copy(
                src_ref=buf.at[0], dst_ref=buf.at[1],
                send_sem=ssem.at[0], recv_sem=rsem.at[1],
                device_id=(tgt,), device_id_type=pl.DeviceIdType.MESH,
            )
            rdma.start()
            inflight[ci] = rdma
            prevprev[ci] = None

        for s in range(N_DEV - 1):
            ss = s % 2
            rs = (s + 1) % 2
            p = {}
            for ci, (buf, ssem, rsem, tgt, lo, is_cw) in enumerate(chains):
                p[ci] = pc(arriving(s, is_cw), lo)
            for ci, (buf, ssem, rsem, tgt, lo, is_cw) in enumerate(chains):
                cur = inflight[ci]
                cur.wait_recv()
                if prevprev[ci] is not None:
                    prevprev[ci].wait_send()
                acc = buf[rs].astype(jnp.float32) + p[ci]
                if s < N_DEV - 2:
                    buf[rs] = acc.astype(jnp.bfloat16)
                    nxt = pltpu.make_async_remote_copy(
                        src_ref=buf.at[rs], dst_ref=buf.at[ss],
                        send_sem=ssem.at[rs], recv_sem=rsem.at[ss],
                        device_id=(tgt,), device_id_type=pl.DeviceIdType.MESH,
                    )
                    nxt.start()
                    prevprev[ci] = cur
                    inflight[ci] = nxt
                else:
                    out_ref[:, lo:lo + ns] = _gelu(acc)

        for ci in range(len(chains)):
            inflight[ci].wait_send()

    return pl.pallas_call(
        body,
        out_shape=jax.ShapeDtypeStruct((m_per, n), jnp.float32),
        in_specs=[
            pl.BlockSpec(memory_space=pltpu.VMEM),
            pl.BlockSpec(memory_space=pltpu.VMEM),
        ],
        out_specs=pl.BlockSpec(memory_space=pltpu.VMEM),
        scratch_shapes=(
            [pltpu.VMEM((2, m_per, ns), jnp.bfloat16)] * (2 * Q)
            + [pltpu.SemaphoreType.DMA((2,))] * (4 * Q)
        ),
        compiler_params=pltpu.CompilerParams(
            collective_id=0,
            vmem_limit_bytes=100 * 1024 * 1024,
        ),
    )(x, w_mat)
